# baseline (speedup 1.0000x reference)
import functools

import jax
import jax.numpy as jnp
from jax import lax
from jax.experimental import pallas as pl
from jax.experimental.pallas import tpu as pltpu
from jax.experimental.pallas import tpu_sc as plsc

BN_EPS = 1e-3
LN_EPS = 1e-3

_NC = 2
_NS = 16
_NW = _NC * _NS
_LANES = 16


def _sc_pool_kernel(B, L, V2, D, CB, NCH):
    G = CB * L
    RPW = B // _NW
    W2 = 2 * D
    NB = 2

    mesh = plsc.VectorSubcoreMesh(core_axis_name="c", subcore_axis_name="s")

    @functools.partial(
        pl.kernel,
        out_type=jax.ShapeDtypeStruct((B, D), jnp.float32),
        mesh=mesh,
        scratch_types=[
            pltpu.VMEM((NCH, G), jnp.int32),
            pltpu.VMEM((G, W2), jnp.float32),
            pltpu.VMEM((G, W2), jnp.float32),
            pltpu.VMEM((G * _LANES,), jnp.float32),
            pltpu.VMEM((G * _LANES,), jnp.float32),
            pltpu.VMEM((RPW, D), jnp.float32),
            pltpu.SemaphoreType.DMA,
            pltpu.SemaphoreType.DMA,
            pltpu.SemaphoreType.DMA,
            pltpu.SemaphoreType.DMA,
        ],
    )
    def sc_pool(widx_hbm, hrep_hbm, table_hbm, out_hbm, idx_v, rows0, rows1,
                h0, h1, acc_v, gs0, gs1, os0, os1):
        wid = lax.axis_index("s") * _NC + lax.axis_index("c")
        pltpu.sync_copy(widx_hbm.at[pl.ds(wid * NCH, NCH)], idx_v)

        nvec = D // _LANES
        rows = (rows0, rows1)
        hbufs = (h0, h1)
        gsems = (gs0, gs1)
        osems = (os0, os1)

        def start(g, t):
            pltpu.async_copy(table_hbm.at[idx_v.at[g]], rows[t], gsems[t])
            pltpu.async_copy(hrep_hbm.at[wid * NCH + g], hbufs[t], osems[t])

        for t in range(NB):
            start(t, t)

        def outer(i, carry):
            jj = i * NB
            for t in range(NB):
                g = jj + t
                pltpu.make_async_copy(table_hbm.at[idx_v.at[g]], rows[t],
                                      gsems[t]).wait()
                pltpu.make_async_copy(hrep_hbm.at[wid * NCH + g], hbufs[t],
                                      osems[t]).wait()

                def red_body(l, accs, t=t):
                    new = []
                    for bb in range(CB):
                        tok = bb * L + l
                        hv = hbufs[t][pl.ds(tok * _LANES, _LANES)]
                        for d in range(nvec):
                            lo = rows[t][tok, pl.ds(d * _LANES, _LANES)]
                            hi = rows[t][tok, pl.ds(D + d * _LANES, _LANES)]
                            new.append(
                                accs[bb * nvec + d] + lo + hv * (hi - lo))
                    return tuple(new)

                zero = jnp.zeros((_LANES,), jnp.float32)
                accs = lax.fori_loop(0, L, red_body, (zero,) * (CB * nvec))
                for bb in range(CB):
                    for d in range(nvec):
                        acc_v[g * CB + bb, pl.ds(d * _LANES, _LANES)] = (
                            accs[bb * nvec + d])

                @pl.when(g + NB < NCH)
                def _(g=g, t=t):
                    start(g + NB, t)
            return carry

        lax.fori_loop(0, NCH // NB, outer, 0)
        pltpu.sync_copy(acc_v, out_hbm.at[pl.ds(wid * RPW, RPW)])

    return sc_pool


def _tc_transpose_body(x1_ref, x2_ref, o_ref):
    eye = jnp.eye(x1_ref.shape[0], dtype=jnp.float32)
    dn = (((0,), (0,)), ((), ()))
    o_ref[:, 0:64] = jax.lax.dot_general(
        x1_ref[...], eye, dn, preferred_element_type=jnp.float32)
    o_ref[:, 64:128] = jax.lax.dot_general(
        x2_ref[...], eye, dn, preferred_element_type=jnp.float32)


def _tc_transpose(tt, P, R, D):
    V = tt.shape[1]
    nblocks = P // R
    last = (V + R - 1) // R - 1

    return pl.pallas_call(
        _tc_transpose_body,
        grid=(nblocks,),
        in_specs=[
            pl.BlockSpec((D, R), lambda j: (0, j)),
            pl.BlockSpec((D, R), lambda j: (0, jnp.minimum(nblocks + j, last))),
        ],
        out_specs=pl.BlockSpec((R, 2 * D), lambda j: (j, 0)),
        out_shape=jax.ShapeDtypeStruct((P, 2 * D), jnp.float32),
    )(tt, tt)


def _tc_dense_body(x_ref, w_ref, b_ref, bng_ref, bnb_ref, bnm_ref, bnv_ref,
                   lng_ref, lnb_ref, inv_l_ref, o_ref):
    x = x_ref[...] * inv_l_ref[0, 0]
    y = jnp.dot(x, w_ref[...], preferred_element_type=jnp.float32,
                precision=jax.lax.Precision.HIGHEST)
    y = y + b_ref[...]
    y = bng_ref[...] * (y - bnm_ref[...]) * jax.lax.rsqrt(bnv_ref[...] + BN_EPS)
    y = y + bnb_ref[...]
    mu = jnp.mean(y, axis=-1, keepdims=True)
    var = jnp.mean((y - mu) ** 2, axis=-1, keepdims=True)
    o_ref[...] = lng_ref[...] * (y - mu) * jax.lax.rsqrt(var + LN_EPS) + lnb_ref[...]


def kernel(inputs, table, W, b, bn_gamma, bn_beta, bn_mean, bn_var, ln_gamma, ln_beta):
    B, L = inputs.shape
    V, D = table.shape

    CB = max(1, 128 // L)
    while (B // _NW) % CB:
        CB -= 1
    NCH = B // (_NW * CB)
    G = CB * L

    P = 524288
    tok = inputs.astype(jnp.int32).reshape(_NW * NCH, G)
    widx = tok & (P - 1)
    h = (tok >> 19).astype(jnp.float32)
    hrep = jnp.broadcast_to(h[:, :, None], (_NW * NCH, G, _LANES))
    hrep = hrep.reshape(_NW * NCH, G * _LANES)
    table2 = _tc_transpose(table.T, P, 4096, D)

    sums = _sc_pool_kernel(B, L, P, D, CB, NCH)(widx, hrep, table2)

    row = lambda v: v.reshape(1, D).astype(jnp.float32)
    inv_l = jnp.full((1, 1), 1.0 / L, dtype=jnp.float32)
    out = pl.pallas_call(
        _tc_dense_body,
        out_shape=jax.ShapeDtypeStruct((B, D), jnp.float32),
    )(sums, W, row(b), row(bn_gamma), row(bn_beta), row(bn_mean), row(bn_var),
      row(ln_gamma), row(ln_beta), inv_l)
    return out


# speedup vs baseline: 1.1000x; 1.1000x over previous
import functools

import jax
import jax.numpy as jnp
from jax import lax
from jax.experimental import pallas as pl
from jax.experimental.pallas import tpu as pltpu
from jax.experimental.pallas import tpu_sc as plsc

BN_EPS = 1e-3
LN_EPS = 1e-3

_NC = 2
_NS = 16
_NW = _NC * _NS
_LANES = 16


def _sc_pool_kernel(B, L, V2, D, CB, NCH):
    G = CB * L
    RPW = B // _NW
    W2 = 2 * D
    NB = 2

    mesh = plsc.VectorSubcoreMesh(core_axis_name="c", subcore_axis_name="s")

    @functools.partial(
        pl.kernel,
        out_type=jax.ShapeDtypeStruct((B, D), jnp.float32),
        mesh=mesh,
        scratch_types=[
            pltpu.VMEM((NCH, G), jnp.int32),
            pltpu.VMEM((G, W2), jnp.float32),
            pltpu.VMEM((G, W2), jnp.float32),
            pltpu.VMEM((G * _LANES,), jnp.float32),
            pltpu.VMEM((G * _LANES,), jnp.float32),
            pltpu.VMEM((RPW, D), jnp.float32),
            pltpu.SemaphoreType.DMA,
            pltpu.SemaphoreType.DMA,
            pltpu.SemaphoreType.DMA,
            pltpu.SemaphoreType.DMA,
        ],
    )
    def sc_pool(widx_hbm, hrep_hbm, table_hbm, out_hbm, idx_v, rows0, rows1,
                h0, h1, acc_v, gs0, gs1, os0, os1):
        wid = lax.axis_index("s") * _NC + lax.axis_index("c")
        pltpu.sync_copy(widx_hbm.at[pl.ds(wid * NCH, NCH)], idx_v)

        nvec = D // _LANES
        rows = (rows0, rows1)
        hbufs = (h0, h1)
        gsems = (gs0, gs1)
        osems = (os0, os1)

        def start(g, t):
            pltpu.async_copy(table_hbm.at[idx_v.at[g]], rows[t], gsems[t])
            pltpu.async_copy(hrep_hbm.at[wid * NCH + g], hbufs[t], osems[t])

        for t in range(NB):
            start(t, t)

        def outer(i, carry):
            jj = i * NB
            for t in range(NB):
                g = jj + t
                pltpu.make_async_copy(table_hbm.at[idx_v.at[g]], rows[t],
                                      gsems[t]).wait()
                pltpu.make_async_copy(hrep_hbm.at[wid * NCH + g], hbufs[t],
                                      osems[t]).wait()

                def red_body(l, accs, t=t):
                    new = []
                    for bb in range(CB):
                        tok = bb * L + l
                        hv = hbufs[t][pl.ds(tok * _LANES, _LANES)]
                        for d in range(nvec):
                            lo = rows[t][tok, pl.ds(d * _LANES, _LANES)]
                            hi = rows[t][tok, pl.ds(D + d * _LANES, _LANES)]
                            new.append(
                                accs[bb * nvec + d] + lo + hv * (hi - lo))
                    return tuple(new)

                zero = jnp.zeros((_LANES,), jnp.float32)
                accs = lax.fori_loop(0, L, red_body, (zero,) * (CB * nvec))
                for bb in range(CB):
                    for d in range(nvec):
                        acc_v[g * CB + bb, pl.ds(d * _LANES, _LANES)] = (
                            accs[bb * nvec + d])

                @pl.when(g + NB < NCH)
                def _(g=g, t=t):
                    start(g + NB, t)
            return carry

        lax.fori_loop(0, NCH // NB, outer, 0)
        pltpu.sync_copy(acc_v, out_hbm.at[pl.ds(wid * RPW, RPW)])

    return sc_pool


def _tc_transpose_body(x1_ref, x2_ref, o_ref):
    eye = jnp.eye(x1_ref.shape[0], dtype=jnp.float32)
    dn = (((0,), (0,)), ((), ()))
    o_ref[:, 0:64] = jax.lax.dot_general(
        x1_ref[...], eye, dn, preferred_element_type=jnp.float32)
    o_ref[:, 64:128] = jax.lax.dot_general(
        x2_ref[...], eye, dn, preferred_element_type=jnp.float32)


def _tc_transpose(tt, P, R, D):
    V = tt.shape[1]
    nblocks = P // R
    last = (V + R - 1) // R - 1

    return pl.pallas_call(
        _tc_transpose_body,
        grid=(nblocks,),
        in_specs=[
            pl.BlockSpec((D, R), lambda j: (0, j)),
            pl.BlockSpec((D, R), lambda j: (0, jnp.minimum(nblocks + j, last))),
        ],
        out_specs=pl.BlockSpec((R, 2 * D), lambda j: (j, 0)),
        out_shape=jax.ShapeDtypeStruct((P, 2 * D), jnp.float32),
    )(tt, tt)


def _tc_dense_body(x_ref, w_ref, b_ref, bng_ref, bnb_ref, bnm_ref, bnv_ref,
                   lng_ref, lnb_ref, inv_l_ref, o_ref):
    x = x_ref[...] * inv_l_ref[0, 0]
    y = jnp.dot(x, w_ref[...], preferred_element_type=jnp.float32,
                precision=jax.lax.Precision.HIGHEST)
    y = y + b_ref[...]
    y = bng_ref[...] * (y - bnm_ref[...]) * jax.lax.rsqrt(bnv_ref[...] + BN_EPS)
    y = y + bnb_ref[...]
    mu = jnp.mean(y, axis=-1, keepdims=True)
    var = jnp.mean((y - mu) ** 2, axis=-1, keepdims=True)
    o_ref[...] = lng_ref[...] * (y - mu) * jax.lax.rsqrt(var + LN_EPS) + lnb_ref[...]


def kernel(inputs, table, W, b, bn_gamma, bn_beta, bn_mean, bn_var, ln_gamma, ln_beta):
    B, L = inputs.shape
    V, D = table.shape

    CB = max(1, 128 // L)
    while (B // _NW) % CB:
        CB -= 1
    NCH = B // (_NW * CB)
    G = CB * L

    P = 524288
    tok = inputs.astype(jnp.int32).reshape(_NW * NCH, G)
    widx = tok & (P - 1)
    h = (tok >> 19).astype(jnp.float32)
    hrep = jnp.broadcast_to(h[:, :, None], (_NW * NCH, G, _LANES))
    hrep = hrep.reshape(_NW * NCH, G * _LANES)
    table2 = _tc_transpose(table.T, P, 8192, D)

    sums = _sc_pool_kernel(B, L, P, D, CB, NCH)(widx, hrep, table2)

    row = lambda v: v.reshape(1, D).astype(jnp.float32)
    inv_l = jnp.full((1, 1), 1.0 / L, dtype=jnp.float32)
    out = pl.pallas_call(
        _tc_dense_body,
        out_shape=jax.ShapeDtypeStruct((B, D), jnp.float32),
    )(sums, W, row(b), row(bn_gamma), row(bn_beta), row(bn_mean), row(bn_var),
      row(ln_gamma), row(ln_beta), inv_l)
    return out


# speedup vs baseline: 1.1470x; 1.0427x over previous
import functools

import jax
import jax.numpy as jnp
from jax import lax
from jax.experimental import pallas as pl
from jax.experimental.pallas import tpu as pltpu
from jax.experimental.pallas import tpu_sc as plsc

BN_EPS = 1e-3
LN_EPS = 1e-3

_NC = 2
_NS = 16
_NW = _NC * _NS
_LANES = 16


def _sc_pool_kernel(B, L, V2, D, CB, NCH):
    G = CB * L
    RPW = B // _NW
    W2 = 2 * D
    NB = 2

    mesh = plsc.VectorSubcoreMesh(core_axis_name="c", subcore_axis_name="s")

    @functools.partial(
        pl.kernel,
        out_type=jax.ShapeDtypeStruct((B, D), jnp.float32),
        mesh=mesh,
        scratch_types=[
            pltpu.VMEM((NCH, G), jnp.int32),
            pltpu.VMEM((G, W2), jnp.float32),
            pltpu.VMEM((G, W2), jnp.float32),
            pltpu.VMEM((G * _LANES,), jnp.float32),
            pltpu.VMEM((G * _LANES,), jnp.float32),
            pltpu.VMEM((RPW, D), jnp.float32),
            pltpu.SemaphoreType.DMA,
            pltpu.SemaphoreType.DMA,
            pltpu.SemaphoreType.DMA,
            pltpu.SemaphoreType.DMA,
        ],
    )
    def sc_pool(widx_hbm, hrep_hbm, table_hbm, out_hbm, idx_v, rows0, rows1,
                h0, h1, acc_v, gs0, gs1, os0, os1):
        wid = lax.axis_index("s") * _NC + lax.axis_index("c")
        pltpu.sync_copy(widx_hbm.at[pl.ds(wid * NCH, NCH)], idx_v)

        nvec = D // _LANES
        rows = (rows0, rows1)
        hbufs = (h0, h1)
        gsems = (gs0, gs1)
        osems = (os0, os1)

        def start(g, t):
            pltpu.async_copy(table_hbm.at[idx_v.at[g]], rows[t], gsems[t])
            pltpu.async_copy(hrep_hbm.at[wid * NCH + g], hbufs[t], osems[t])

        for t in range(NB):
            start(t, t)

        def outer(i, carry):
            jj = i * NB
            for t in range(NB):
                g = jj + t
                pltpu.make_async_copy(table_hbm.at[idx_v.at[g]], rows[t],
                                      gsems[t]).wait()
                pltpu.make_async_copy(hrep_hbm.at[wid * NCH + g], hbufs[t],
                                      osems[t]).wait()

                def red_body(l, accs, t=t):
                    new = []
                    for bb in range(CB):
                        tok = bb * L + l
                        hv = hbufs[t][pl.ds(tok * _LANES, _LANES)]
                        for d in range(nvec):
                            lo = rows[t][tok, pl.ds(d * _LANES, _LANES)]
                            hi = rows[t][tok, pl.ds(D + d * _LANES, _LANES)]
                            new.append(
                                accs[bb * nvec + d] + lo + hv * (hi - lo))
                    return tuple(new)

                zero = jnp.zeros((_LANES,), jnp.float32)
                accs = lax.fori_loop(0, L, red_body, (zero,) * (CB * nvec))
                for bb in range(CB):
                    for d in range(nvec):
                        acc_v[g * CB + bb, pl.ds(d * _LANES, _LANES)] = (
                            accs[bb * nvec + d])

                @pl.when(g + NB < NCH)
                def _(g=g, t=t):
                    start(g + NB, t)
            return carry

        lax.fori_loop(0, NCH // NB, outer, 0)
        pltpu.sync_copy(acc_v, out_hbm.at[pl.ds(wid * RPW, RPW)])

    return sc_pool


def _tc_transpose_body(x1_ref, x2_ref, o_ref):
    eye = jnp.eye(x1_ref.shape[0], dtype=jnp.float32)
    dn = (((0,), (0,)), ((), ()))
    o_ref[:, 0:64] = jax.lax.dot_general(
        x1_ref[...], eye, dn, preferred_element_type=jnp.float32)
    o_ref[:, 64:128] = jax.lax.dot_general(
        x2_ref[...], eye, dn, preferred_element_type=jnp.float32)


def _tc_transpose(tt, P, R, D):
    V = tt.shape[1]
    nblocks = P // R
    last = (V + R - 1) // R - 1

    return pl.pallas_call(
        _tc_transpose_body,
        grid=(nblocks,),
        in_specs=[
            pl.BlockSpec((D, R), lambda j: (0, j)),
            pl.BlockSpec((D, R), lambda j: (0, jnp.minimum(nblocks + j, last))),
        ],
        out_specs=pl.BlockSpec((R, 2 * D), lambda j: (j, 0)),
        out_shape=jax.ShapeDtypeStruct((P, 2 * D), jnp.float32),
    )(tt, tt)


def _tc_dense_body(x_ref, w_ref, b_ref, bng_ref, bnb_ref, bnm_ref, bnv_ref,
                   lng_ref, lnb_ref, inv_l_ref, o_ref):
    x = x_ref[...] * inv_l_ref[0, 0]
    y = jnp.dot(x, w_ref[...], preferred_element_type=jnp.float32,
                precision=jax.lax.Precision.HIGHEST)
    y = y + b_ref[...]
    y = bng_ref[...] * (y - bnm_ref[...]) * jax.lax.rsqrt(bnv_ref[...] + BN_EPS)
    y = y + bnb_ref[...]
    mu = jnp.mean(y, axis=-1, keepdims=True)
    var = jnp.mean((y - mu) ** 2, axis=-1, keepdims=True)
    o_ref[...] = lng_ref[...] * (y - mu) * jax.lax.rsqrt(var + LN_EPS) + lnb_ref[...]


def kernel(inputs, table, W, b, bn_gamma, bn_beta, bn_mean, bn_var, ln_gamma, ln_beta):
    B, L = inputs.shape
    V, D = table.shape

    CB = max(1, 128 // L)
    while (B // _NW) % CB:
        CB -= 1
    NCH = B // (_NW * CB)
    G = CB * L

    P = 524288
    tok = inputs.astype(jnp.int32).reshape(_NW * NCH, G)
    widx = tok & (P - 1)
    h = (tok >> 19).astype(jnp.float32)
    hrep = jnp.broadcast_to(h[:, :, None], (_NW * NCH, G, _LANES))
    hrep = hrep.reshape(_NW * NCH, G * _LANES)
    table2 = _tc_transpose(table.T, P, 16384, D)

    sums = _sc_pool_kernel(B, L, P, D, CB, NCH)(widx, hrep, table2)

    row = lambda v: v.reshape(1, D).astype(jnp.float32)
    inv_l = jnp.full((1, 1), 1.0 / L, dtype=jnp.float32)
    out = pl.pallas_call(
        _tc_dense_body,
        out_shape=jax.ShapeDtypeStruct((B, D), jnp.float32),
    )(sums, W, row(b), row(bn_gamma), row(bn_beta), row(bn_mean), row(bn_var),
      row(ln_gamma), row(ln_beta), inv_l)
    return out


# speedup vs baseline: 1.3150x; 1.1465x over previous
import functools

import jax
import jax.numpy as jnp
from jax import lax
from jax.experimental import pallas as pl
from jax.experimental.pallas import tpu as pltpu
from jax.experimental.pallas import tpu_sc as plsc

BN_EPS = 1e-3
LN_EPS = 1e-3

_NC = 2
_NS = 16
_NW = _NC * _NS
_LANES = 16


def _sc_pool_kernel(B, L, V2, D, CB, NCH):
    G = CB * L
    RPW = B // _NW
    NB = 2

    mesh = plsc.VectorSubcoreMesh(core_axis_name="c", subcore_axis_name="s")

    @functools.partial(
        pl.kernel,
        out_type=jax.ShapeDtypeStruct((B, D), jnp.float32),
        mesh=mesh,
        scratch_types=[
            pltpu.VMEM((NCH, G), jnp.int32),
            pltpu.VMEM((G, D), jnp.float32),
            pltpu.VMEM((G, D), jnp.float32),
            pltpu.VMEM((RPW, D), jnp.float32),
            pltpu.SemaphoreType.DMA,
            pltpu.SemaphoreType.DMA,
        ],
        compiler_params=pltpu.CompilerParams(use_tc_tiling_on_sc=False),
    )
    def sc_pool(idx_hbm, table_hbm, out_hbm, idx_v, rows0, rows1, acc_v,
                gs0, gs1):
        wid = lax.axis_index("s") * _NC + lax.axis_index("c")
        pltpu.sync_copy(idx_hbm.at[pl.ds(wid * NCH, NCH)], idx_v)

        nvec = D // _LANES
        rows = (rows0, rows1)
        gsems = (gs0, gs1)

        def start(g, t):
            pltpu.async_copy(table_hbm.at[idx_v.at[g]], rows[t], gsems[t])

        for t in range(NB):
            start(t, t)

        def outer(i, carry):
            jj = i * NB
            for t in range(NB):
                g = jj + t
                pltpu.make_async_copy(table_hbm.at[idx_v.at[g]], rows[t],
                                      gsems[t]).wait()

                def red_body(l, accs, t=t):
                    new = []
                    for bb in range(CB):
                        tok = bb * L + l
                        for d in range(nvec):
                            new.append(
                                accs[bb * nvec + d]
                                + rows[t][tok, pl.ds(d * _LANES, _LANES)])
                    return tuple(new)

                zero = jnp.zeros((_LANES,), jnp.float32)
                accs = lax.fori_loop(0, L, red_body, (zero,) * (CB * nvec))
                for bb in range(CB):
                    for d in range(nvec):
                        acc_v[g * CB + bb, pl.ds(d * _LANES, _LANES)] = (
                            accs[bb * nvec + d])

                @pl.when(g + NB < NCH)
                def _(g=g, t=t):
                    start(g + NB, t)
            return carry

        lax.fori_loop(0, NCH // NB, outer, 0)
        pltpu.sync_copy(acc_v, out_hbm.at[pl.ds(wid * RPW, RPW)])

    return sc_pool


def _tc_transpose_body(x1_ref, x2_ref, o_ref):
    eye = jnp.eye(x1_ref.shape[0], dtype=jnp.float32)
    dn = (((0,), (0,)), ((), ()))
    o_ref[:, 0:64] = jax.lax.dot_general(
        x1_ref[...], eye, dn, preferred_element_type=jnp.float32)
    o_ref[:, 64:128] = jax.lax.dot_general(
        x2_ref[...], eye, dn, preferred_element_type=jnp.float32)


def _tc_transpose(tt, P, R, D):
    V = tt.shape[1]
    nblocks = P // R
    last = (V + R - 1) // R - 1

    return pl.pallas_call(
        _tc_transpose_body,
        grid=(nblocks,),
        in_specs=[
            pl.BlockSpec((D, R), lambda j: (0, j)),
            pl.BlockSpec((D, R), lambda j: (0, jnp.minimum(nblocks + j, last))),
        ],
        out_specs=pl.BlockSpec((R, 2 * D), lambda j: (j, 0)),
        out_shape=jax.ShapeDtypeStruct((P, 2 * D), jnp.float32),
    )(tt, tt)


def _tc_dense_body(x_ref, w_ref, b_ref, bng_ref, bnb_ref, bnm_ref, bnv_ref,
                   lng_ref, lnb_ref, inv_l_ref, o_ref):
    x = x_ref[...] * inv_l_ref[0, 0]
    y = jnp.dot(x, w_ref[...], preferred_element_type=jnp.float32,
                precision=jax.lax.Precision.HIGHEST)
    y = y + b_ref[...]
    y = bng_ref[...] * (y - bnm_ref[...]) * jax.lax.rsqrt(bnv_ref[...] + BN_EPS)
    y = y + bnb_ref[...]
    mu = jnp.mean(y, axis=-1, keepdims=True)
    var = jnp.mean((y - mu) ** 2, axis=-1, keepdims=True)
    o_ref[...] = lng_ref[...] * (y - mu) * jax.lax.rsqrt(var + LN_EPS) + lnb_ref[...]


def kernel(inputs, table, W, b, bn_gamma, bn_beta, bn_mean, bn_var, ln_gamma, ln_beta):
    B, L = inputs.shape
    V, D = table.shape

    CB = max(1, 128 // L)
    while (B // _NW) % CB:
        CB -= 1
    NCH = B // (_NW * CB)
    G = CB * L

    P = 524288
    tok = inputs.astype(jnp.int32).reshape(_NW * NCH, G)
    widx = ((tok & (P - 1)) << 1) | (tok >> 19)
    table2 = _tc_transpose(table.T, P, 16384, D)
    table3 = table2.reshape(2 * P, D)

    sums = _sc_pool_kernel(B, L, 2 * P, D, CB, NCH)(widx, table3)

    row = lambda v: v.reshape(1, D).astype(jnp.float32)
    inv_l = jnp.full((1, 1), 1.0 / L, dtype=jnp.float32)
    out = pl.pallas_call(
        _tc_dense_body,
        out_shape=jax.ShapeDtypeStruct((B, D), jnp.float32),
    )(sums, W, row(b), row(bn_gamma), row(bn_beta), row(bn_mean), row(bn_var),
      row(ln_gamma), row(ln_beta), inv_l)
    return out


# speedup vs baseline: 1.3928x; 1.0591x over previous
import functools

import jax
import jax.numpy as jnp
from jax import lax
from jax.experimental import pallas as pl
from jax.experimental.pallas import tpu as pltpu
from jax.experimental.pallas import tpu_sc as plsc

BN_EPS = 1e-3
LN_EPS = 1e-3

_NC = 2
_NS = 16
_NW = _NC * _NS
_LANES = 16


def _sc_pool_kernel(B, L, V2, D, CB, NCH):
    G = CB * L
    RPW = B // _NW
    NB = 4

    mesh = plsc.VectorSubcoreMesh(core_axis_name="c", subcore_axis_name="s")

    @functools.partial(
        pl.kernel,
        out_type=jax.ShapeDtypeStruct((B, D), jnp.float32),
        mesh=mesh,
        scratch_types=[
            pltpu.VMEM((NCH, G), jnp.int32),
            pltpu.VMEM((G, D), jnp.float32),
            pltpu.VMEM((G, D), jnp.float32),
            pltpu.VMEM((G, D), jnp.float32),
            pltpu.VMEM((G, D), jnp.float32),
            pltpu.VMEM((RPW, D), jnp.float32),
            pltpu.SemaphoreType.DMA,
            pltpu.SemaphoreType.DMA,
            pltpu.SemaphoreType.DMA,
            pltpu.SemaphoreType.DMA,
        ],
        compiler_params=pltpu.CompilerParams(use_tc_tiling_on_sc=False),
    )
    def sc_pool(idx_hbm, table_hbm, out_hbm, idx_v, rows0, rows1, rows2,
                rows3, acc_v, gs0, gs1, gs2, gs3):
        wid = lax.axis_index("s") * _NC + lax.axis_index("c")
        pltpu.sync_copy(idx_hbm.at[pl.ds(wid * NCH, NCH)], idx_v)

        nvec = D // _LANES
        rows = (rows0, rows1, rows2, rows3)
        gsems = (gs0, gs1, gs2, gs3)

        def start(g, t):
            pltpu.async_copy(table_hbm.at[idx_v.at[g]], rows[t], gsems[t])

        for t in range(NB):
            start(t, t)

        def outer(i, carry):
            jj = i * NB
            for t in range(NB):
                g = jj + t
                pltpu.make_async_copy(table_hbm.at[idx_v.at[g]], rows[t],
                                      gsems[t]).wait()

                def red_body(l, accs, t=t):
                    new = []
                    for bb in range(CB):
                        tok = bb * L + l
                        for d in range(nvec):
                            new.append(
                                accs[bb * nvec + d]
                                + rows[t][tok, pl.ds(d * _LANES, _LANES)])
                    return tuple(new)

                zero = jnp.zeros((_LANES,), jnp.float32)
                accs = lax.fori_loop(0, L, red_body, (zero,) * (CB * nvec))
                for bb in range(CB):
                    for d in range(nvec):
                        acc_v[g * CB + bb, pl.ds(d * _LANES, _LANES)] = (
                            accs[bb * nvec + d])

                @pl.when(g + NB < NCH)
                def _(g=g, t=t):
                    start(g + NB, t)
            return carry

        lax.fori_loop(0, NCH // NB, outer, 0)
        pltpu.sync_copy(acc_v, out_hbm.at[pl.ds(wid * RPW, RPW)])

    return sc_pool


def _tc_transpose_body(x1_ref, x2_ref, o_ref):
    eye = jnp.eye(x1_ref.shape[0], dtype=jnp.float32)
    dn = (((0,), (0,)), ((), ()))
    o_ref[:, 0:64] = jax.lax.dot_general(
        x1_ref[...], eye, dn, preferred_element_type=jnp.float32)
    o_ref[:, 64:128] = jax.lax.dot_general(
        x2_ref[...], eye, dn, preferred_element_type=jnp.float32)


def _tc_transpose(tt, P, R, D):
    V = tt.shape[1]
    nblocks = P // R
    last = (V + R - 1) // R - 1

    return pl.pallas_call(
        _tc_transpose_body,
        grid=(nblocks,),
        in_specs=[
            pl.BlockSpec((D, R), lambda j: (0, j)),
            pl.BlockSpec((D, R), lambda j: (0, jnp.minimum(nblocks + j, last))),
        ],
        out_specs=pl.BlockSpec((R, 2 * D), lambda j: (j, 0)),
        out_shape=jax.ShapeDtypeStruct((P, 2 * D), jnp.float32),
    )(tt, tt)


def _tc_dense_body(x_ref, w_ref, b_ref, bng_ref, bnb_ref, bnm_ref, bnv_ref,
                   lng_ref, lnb_ref, inv_l_ref, o_ref):
    x = x_ref[...] * inv_l_ref[0, 0]
    y = jnp.dot(x, w_ref[...], preferred_element_type=jnp.float32,
                precision=jax.lax.Precision.HIGHEST)
    y = y + b_ref[...]
    y = bng_ref[...] * (y - bnm_ref[...]) * jax.lax.rsqrt(bnv_ref[...] + BN_EPS)
    y = y + bnb_ref[...]
    mu = jnp.mean(y, axis=-1, keepdims=True)
    var = jnp.mean((y - mu) ** 2, axis=-1, keepdims=True)
    o_ref[...] = lng_ref[...] * (y - mu) * jax.lax.rsqrt(var + LN_EPS) + lnb_ref[...]


def kernel(inputs, table, W, b, bn_gamma, bn_beta, bn_mean, bn_var, ln_gamma, ln_beta):
    B, L = inputs.shape
    V, D = table.shape

    CB = max(1, 128 // L)
    while (B // _NW) % CB:
        CB -= 1
    NCH = B // (_NW * CB)
    G = CB * L

    P = 524288
    tok = inputs.astype(jnp.int32).reshape(_NW * NCH, G)
    widx = ((tok & (P - 1)) << 1) | (tok >> 19)
    table2 = _tc_transpose(table.T, P, 16384, D)
    table3 = table2.reshape(2 * P, D)

    sums = _sc_pool_kernel(B, L, 2 * P, D, CB, NCH)(widx, table3)

    row = lambda v: v.reshape(1, D).astype(jnp.float32)
    inv_l = jnp.full((1, 1), 1.0 / L, dtype=jnp.float32)
    out = pl.pallas_call(
        _tc_dense_body,
        out_shape=jax.ShapeDtypeStruct((B, D), jnp.float32),
    )(sums, W, row(b), row(bn_gamma), row(bn_beta), row(bn_mean), row(bn_var),
      row(ln_gamma), row(ln_beta), inv_l)
    return out


# speedup vs baseline: 1.4139x; 1.0152x over previous
import functools

import jax
import jax.numpy as jnp
from jax import lax
from jax.experimental import pallas as pl
from jax.experimental.pallas import tpu as pltpu
from jax.experimental.pallas import tpu_sc as plsc

BN_EPS = 1e-3
LN_EPS = 1e-3

_NC = 2
_NS = 16
_NW = _NC * _NS
_LANES = 16


def _sc_pool_kernel(B, L, V2, D, CB, NCH):
    G = CB * L
    RPW = B // _NW
    NB = 8

    mesh = plsc.VectorSubcoreMesh(core_axis_name="c", subcore_axis_name="s")

    @functools.partial(
        pl.kernel,
        out_type=jax.ShapeDtypeStruct((B, D), jnp.float32),
        mesh=mesh,
        scratch_types=[
            pltpu.VMEM((NCH, G), jnp.int32),
        ] + [pltpu.VMEM((G, D), jnp.float32)] * 8
          + [pltpu.VMEM((RPW, D), jnp.float32)]
          + [pltpu.SemaphoreType.DMA] * 8,
        compiler_params=pltpu.CompilerParams(use_tc_tiling_on_sc=False),
    )
    def sc_pool(idx_hbm, table_hbm, out_hbm, idx_v, rows0, rows1, rows2,
                rows3, rows4, rows5, rows6, rows7, acc_v,
                gs0, gs1, gs2, gs3, gs4, gs5, gs6, gs7):
        wid = lax.axis_index("s") * _NC + lax.axis_index("c")
        pltpu.sync_copy(idx_hbm.at[pl.ds(wid * NCH, NCH)], idx_v)

        nvec = D // _LANES
        rows = (rows0, rows1, rows2, rows3, rows4, rows5, rows6, rows7)
        gsems = (gs0, gs1, gs2, gs3, gs4, gs5, gs6, gs7)

        def start(g, t):
            pltpu.async_copy(table_hbm.at[idx_v.at[g]], rows[t], gsems[t])

        for t in range(NB):
            start(t, t)

        def outer(i, carry):
            jj = i * NB
            for t in range(NB):
                g = jj + t
                pltpu.make_async_copy(table_hbm.at[idx_v.at[g]], rows[t],
                                      gsems[t]).wait()

                def red_body(l, accs, t=t):
                    new = []
                    for bb in range(CB):
                        tok = bb * L + l
                        for d in range(nvec):
                            new.append(
                                accs[bb * nvec + d]
                                + rows[t][tok, pl.ds(d * _LANES, _LANES)])
                    return tuple(new)

                zero = jnp.zeros((_LANES,), jnp.float32)
                accs = lax.fori_loop(0, L, red_body, (zero,) * (CB * nvec))
                for bb in range(CB):
                    for d in range(nvec):
                        acc_v[g * CB + bb, pl.ds(d * _LANES, _LANES)] = (
                            accs[bb * nvec + d])

                @pl.when(g + NB < NCH)
                def _(g=g, t=t):
                    start(g + NB, t)
            return carry

        lax.fori_loop(0, NCH // NB, outer, 0)
        pltpu.sync_copy(acc_v, out_hbm.at[pl.ds(wid * RPW, RPW)])

    return sc_pool


def _tc_transpose_body(x1_ref, x2_ref, o_ref):
    eye = jnp.eye(x1_ref.shape[0], dtype=jnp.float32)
    dn = (((0,), (0,)), ((), ()))
    o_ref[:, 0:64] = jax.lax.dot_general(
        x1_ref[...], eye, dn, preferred_element_type=jnp.float32)
    o_ref[:, 64:128] = jax.lax.dot_general(
        x2_ref[...], eye, dn, preferred_element_type=jnp.float32)


def _tc_transpose(tt, P, R, D):
    V = tt.shape[1]
    nblocks = P // R
    last = (V + R - 1) // R - 1

    return pl.pallas_call(
        _tc_transpose_body,
        grid=(nblocks,),
        in_specs=[
            pl.BlockSpec((D, R), lambda j: (0, j)),
            pl.BlockSpec((D, R), lambda j: (0, jnp.minimum(nblocks + j, last))),
        ],
        out_specs=pl.BlockSpec((R, 2 * D), lambda j: (j, 0)),
        out_shape=jax.ShapeDtypeStruct((P, 2 * D), jnp.float32),
    )(tt, tt)


def _tc_dense_body(x_ref, w_ref, b_ref, bng_ref, bnb_ref, bnm_ref, bnv_ref,
                   lng_ref, lnb_ref, inv_l_ref, o_ref):
    x = x_ref[...] * inv_l_ref[0, 0]
    y = jnp.dot(x, w_ref[...], preferred_element_type=jnp.float32,
                precision=jax.lax.Precision.HIGHEST)
    y = y + b_ref[...]
    y = bng_ref[...] * (y - bnm_ref[...]) * jax.lax.rsqrt(bnv_ref[...] + BN_EPS)
    y = y + bnb_ref[...]
    mu = jnp.mean(y, axis=-1, keepdims=True)
    var = jnp.mean((y - mu) ** 2, axis=-1, keepdims=True)
    o_ref[...] = lng_ref[...] * (y - mu) * jax.lax.rsqrt(var + LN_EPS) + lnb_ref[...]


def kernel(inputs, table, W, b, bn_gamma, bn_beta, bn_mean, bn_var, ln_gamma, ln_beta):
    B, L = inputs.shape
    V, D = table.shape

    CB = max(1, 128 // L)
    while (B // _NW) % CB:
        CB -= 1
    NCH = B // (_NW * CB)
    G = CB * L

    P = 524288
    tok = inputs.astype(jnp.int32).reshape(_NW * NCH, G)
    widx = ((tok & (P - 1)) << 1) | (tok >> 19)
    table2 = _tc_transpose(table.T, P, 16384, D)
    table3 = table2.reshape(2 * P, D)

    sums = _sc_pool_kernel(B, L, 2 * P, D, CB, NCH)(widx, table3)

    row = lambda v: v.reshape(1, D).astype(jnp.float32)
    inv_l = jnp.full((1, 1), 1.0 / L, dtype=jnp.float32)
    out = pl.pallas_call(
        _tc_dense_body,
        out_shape=jax.ShapeDtypeStruct((B, D), jnp.float32),
    )(sums, W, row(b), row(bn_gamma), row(bn_beta), row(bn_mean), row(bn_var),
      row(ln_gamma), row(ln_beta), inv_l)
    return out
